# baseline (speedup 1.0000x reference)
import functools

import jax
import jax.numpy as jnp
import numpy as np
from jax import lax
from jax.experimental import pallas as pl
from jax.experimental.pallas import tpu as pltpu
from jax.experimental.pallas import tpu_sc as plsc

_MU = 100.0
_M = 256.0
_NB = 1024
_SHIFT = 32000
_SEP = _NB + _SHIFT
_KS = float(512.0 / np.log2(_M * _MU + 1.0))

_CW = 512
_NSUB = 32

_IDX0 = (127 << 6)
_YCAP = float(_M * _MU + 1.0)


def _build_tables():
    idx = np.arange(1026, dtype=np.int64)
    bits = ((idx + _IDX0) << 17).astype(np.uint32)
    val = _KS * np.log2(bits.view(np.float32).astype(np.float64))
    val = np.minimum(val, 511.999)
    t0 = val[:-1].astype(np.float32)[:1024]
    t1 = ((val[1:] - val[:-1]) / 2.0**17).astype(np.float32)[:1024]
    return t0, t1


_T0, _T1 = _build_tables()


def _tok16(x, c0v, c1v):
    y = jnp.abs(x) * jnp.float32(_MU) + jnp.float32(1.0)
    y = jnp.minimum(y, jnp.float32(_YCAP))
    b = plsc.bitcast(y, jnp.int32)
    i = (b >> 17) - _IDX0
    rf = (b & 0x1FFFF).astype(jnp.float32)
    c0 = plsc.load_gather(c0v, [i])
    c1 = plsc.load_gather(c1v, [i])
    l2k = c1 * rf + c0
    sgn = plsc.bitcast(x, jnp.int32) & jnp.int32(-2147483648)
    sv = plsc.bitcast(plsc.bitcast(l2k, jnp.int32) ^ sgn, jnp.float32)
    v = sv + jnp.float32(512.0 + _SHIFT)
    v = jnp.minimum(v, jnp.float32(_SHIFT + _NB - 1))
    return v.astype(jnp.int32)


def _sc_body(nfull, remc0, remw, t_hbm, a_hbm, c0_hbm, c1_hbm, o_hbm,
             t0, t1, a0, a1, o0, o1, c0v, c1v, si0, si1, so0, so1):
    wid = lax.axis_index("s") * 2 + lax.axis_index("c")
    sep = jnp.full((16,), _SEP, jnp.int32)
    nmine = (nfull - 1 - wid) // _NSUB + 1
    tb, ab, ob = (t0, t1), (a0, a1), (o0, o1)
    sin, sout = (si0, si1), (so0, so1)
    pltpu.sync_copy(c0_hbm, c0v)
    pltpu.sync_copy(c1_hbm, c1v)

    def c0_of(k):
        return (wid + k * _NSUB) * _CW

    def start_in(k, b):
        c0 = c0_of(k)
        pltpu.async_copy(t_hbm.at[:, pl.ds(c0, _CW)], tb[b], sin[b])
        pltpu.async_copy(a_hbm.at[:, pl.ds(c0, _CW)], ab[b], sin[b])

    def wait_in(k, b):
        c0 = c0_of(k)
        pltpu.make_async_copy(t_hbm.at[:, pl.ds(c0, _CW)], tb[b], sin[b]).wait()
        pltpu.make_async_copy(a_hbm.at[:, pl.ds(c0, _CW)], ab[b], sin[b]).wait()

    def start_out(k, b):
        pltpu.async_copy(ob[b], o_hbm.at[:, pl.ds(c0_of(k), _CW)], sout[b])

    def wait_out(k, b):
        pltpu.make_async_copy(
            ob[b], o_hbm.at[:, pl.ds(c0_of(k), _CW)], sout[b]).wait()

    def compute(t_buf, a_buf, o_buf):
        @plsc.parallel_loop(0, _CW // 16)
        def body(j):
            c = j * 16
            for r in range(16):
                o_buf[r, pl.ds(c, 16)] = _tok16(t_buf[r, pl.ds(c, 16)],
                                                c0v, c1v)
            for r in range(8):
                o_buf[17 + r, pl.ds(c, 16)] = _tok16(a_buf[r, pl.ds(c, 16)],
                                                     c0v, c1v)
            o_buf[16, pl.ds(c, 16)] = sep

    def step(k, b):
        wait_in(k, b)

        @pl.when(k >= 2)
        def _():
            wait_out(k - 2, b)

        compute(tb[b], ab[b], ob[b])
        start_out(k, b)

        @pl.when(k + 2 < nmine)
        def _():
            start_in(k + 2, b)

    start_in(0, 0)

    @pl.when(nmine > 1)
    def _():
        start_in(1, 1)

    def pair_body(p, _):
        step(2 * p, 0)

        @pl.when(2 * p + 1 < nmine)
        def _():
            step(2 * p + 1, 1)

        return 0

    lax.fori_loop(0, (nmine + 1) // 2, pair_body, 0)
    wait_out(2 * ((nmine - 1) // 2), 0)

    @pl.when(nmine > 1)
    def _():
        wait_out(nmine - 1 - (nmine % 2), 1)

    if remw:
        @pl.when(wid == 8)
        def _():
            c0 = remc0
            pltpu.sync_copy(t_hbm.at[:, pl.ds(c0, remw)],
                            t0.at[:, pl.ds(0, remw)])
            pltpu.sync_copy(a_hbm.at[:, pl.ds(c0, remw)],
                            a0.at[:, pl.ds(0, remw)])

            @plsc.parallel_loop(0, remw // 16)
            def body(j):
                c = j * 16
                for r in range(16):
                    o0[r, pl.ds(c, 16)] = _tok16(t0[r, pl.ds(c, 16)],
                                                 c0v, c1v)
                for r in range(8):
                    o0[17 + r, pl.ds(c, 16)] = _tok16(a0[r, pl.ds(c, 16)],
                                                      c0v, c1v)
                o0[16, pl.ds(c, 16)] = sep

            pltpu.sync_copy(o0.at[:, pl.ds(0, remw)],
                            o_hbm.at[:, pl.ds(c0, remw)])


def _tok_ref(x):
    mu = jnp.sign(x) * jnp.log(jnp.abs(x) * _MU + 1.0) / np.log(_M * _MU + 1.0)
    v = jnp.floor((jnp.clip(mu, -1.0, 1.0) + 1.0) * (_NB / 2))
    return jnp.clip(v, 0.0, _NB - 1).astype(jnp.int32) + _SHIFT


@jax.jit
def kernel(tensors, actions):
    n = tensors.shape[0]
    nmain = (n // 128) * 128
    nfull = nmain // _CW
    remc0 = nfull * _CW
    remw = nmain - remc0
    mesh = plsc.VectorSubcoreMesh(core_axis_name="c", subcore_axis_name="s")
    run = pl.kernel(
        functools.partial(_sc_body, nfull, remc0, remw),
        out_type=jax.ShapeDtypeStruct((25, n), jnp.int32),
        mesh=mesh,
        compiler_params=pltpu.CompilerParams(needs_layout_passes=False,
                                             use_tc_tiling_on_sc=True),
        scratch_types=[
            pltpu.VMEM((16, _CW), jnp.float32),
            pltpu.VMEM((16, _CW), jnp.float32),
            pltpu.VMEM((8, _CW), jnp.float32),
            pltpu.VMEM((8, _CW), jnp.float32),
            pltpu.VMEM((25, _CW), jnp.int32),
            pltpu.VMEM((25, _CW), jnp.int32),
            pltpu.VMEM((1024,), jnp.float32),
            pltpu.VMEM((1024,), jnp.float32),
            pltpu.SemaphoreType.DMA,
            pltpu.SemaphoreType.DMA,
            pltpu.SemaphoreType.DMA,
            pltpu.SemaphoreType.DMA,
        ],
    )
    out = run(tensors.T, actions.T, jnp.asarray(_T0), jnp.asarray(_T1)).T
    if nmain == n:
        return out
    tt = _tok_ref(tensors[nmain:])
    at = _tok_ref(actions[nmain:])
    sepcol = jnp.full((n - nmain, 1), _SEP, jnp.int32)
    tail = jnp.concatenate([tt, sepcol, at], axis=1)
    return lax.dynamic_update_slice(out, tail, (nmain, 0))


# speedup vs baseline: 1.0786x; 1.0786x over previous
import functools

import jax
import jax.numpy as jnp
import numpy as np
from jax import lax
from jax.experimental import pallas as pl
from jax.experimental.pallas import tpu as pltpu
from jax.experimental.pallas import tpu_sc as plsc

_MU = 100.0
_M = 256.0
_NB = 1024
_SHIFT = 32000
_SEP = _NB + _SHIFT
_KS = float(512.0 / np.log2(_M * _MU + 1.0))

_CW = 1024
_NSUB = 32

_IDX0 = (127 << 6)
_YCAP = float(_M * _MU + 1.0)


def _build_tables():
    idx = np.arange(1026, dtype=np.int64)
    bits = ((idx + _IDX0) << 17).astype(np.uint32)
    val = _KS * np.log2(bits.view(np.float32).astype(np.float64))
    val = np.minimum(val, 511.999)
    t0 = val[:-1].astype(np.float32)[:1024]
    t1 = ((val[1:] - val[:-1]) / 2.0**17).astype(np.float32)[:1024]
    return t0, t1


_T0, _T1 = _build_tables()


def _tok16(x, c0v, c1v):
    y = jnp.abs(x) * jnp.float32(_MU) + jnp.float32(1.0)
    y = jnp.minimum(y, jnp.float32(_YCAP))
    b = plsc.bitcast(y, jnp.int32)
    i = (b >> 17) - _IDX0
    rf = (b & 0x1FFFF).astype(jnp.float32)
    c0 = plsc.load_gather(c0v, [i])
    c1 = plsc.load_gather(c1v, [i])
    l2k = c1 * rf + c0
    sgn = plsc.bitcast(x, jnp.int32) & jnp.int32(-2147483648)
    sv = plsc.bitcast(plsc.bitcast(l2k, jnp.int32) ^ sgn, jnp.float32)
    v = sv + jnp.float32(512.0 + _SHIFT)
    v = jnp.minimum(v, jnp.float32(_SHIFT + _NB - 1))
    return v.astype(jnp.int32)


def _sc_body(nfull, remc0, remw, t_hbm, a_hbm, c0_hbm, c1_hbm, o_hbm,
             t0, t1, a0, a1, o0, o1, c0v, c1v, si0, si1, so0, so1):
    wid = lax.axis_index("s") * 2 + lax.axis_index("c")
    sep = jnp.full((16,), _SEP, jnp.int32)
    nmine = (nfull - 1 - wid) // _NSUB + 1
    tb, ab, ob = (t0, t1), (a0, a1), (o0, o1)
    sin, sout = (si0, si1), (so0, so1)
    pltpu.sync_copy(c0_hbm, c0v)
    pltpu.sync_copy(c1_hbm, c1v)

    def c0_of(k):
        return (wid + k * _NSUB) * _CW

    def start_in(k, b):
        c0 = c0_of(k)
        pltpu.async_copy(t_hbm.at[:, pl.ds(c0, _CW)], tb[b], sin[b])
        pltpu.async_copy(a_hbm.at[:, pl.ds(c0, _CW)], ab[b], sin[b])

    def wait_in(k, b):
        c0 = c0_of(k)
        pltpu.make_async_copy(t_hbm.at[:, pl.ds(c0, _CW)], tb[b], sin[b]).wait()
        pltpu.make_async_copy(a_hbm.at[:, pl.ds(c0, _CW)], ab[b], sin[b]).wait()

    def start_out(k, b):
        pltpu.async_copy(ob[b], o_hbm.at[:, pl.ds(c0_of(k), _CW)], sout[b])

    def wait_out(k, b):
        pltpu.make_async_copy(
            ob[b], o_hbm.at[:, pl.ds(c0_of(k), _CW)], sout[b]).wait()

    def compute(t_buf, a_buf, o_buf):
        @plsc.parallel_loop(0, _CW // 16)
        def body(j):
            c = j * 16
            for r in range(16):
                o_buf[r, pl.ds(c, 16)] = _tok16(t_buf[r, pl.ds(c, 16)],
                                                c0v, c1v)
            for r in range(8):
                o_buf[17 + r, pl.ds(c, 16)] = _tok16(a_buf[r, pl.ds(c, 16)],
                                                     c0v, c1v)
            o_buf[16, pl.ds(c, 16)] = sep

    def step(k, b):
        wait_in(k, b)

        @pl.when(k >= 2)
        def _():
            wait_out(k - 2, b)

        compute(tb[b], ab[b], ob[b])
        start_out(k, b)

        @pl.when(k + 2 < nmine)
        def _():
            start_in(k + 2, b)

    start_in(0, 0)

    @pl.when(nmine > 1)
    def _():
        start_in(1, 1)

    def pair_body(p, _):
        step(2 * p, 0)

        @pl.when(2 * p + 1 < nmine)
        def _():
            step(2 * p + 1, 1)

        return 0

    lax.fori_loop(0, (nmine + 1) // 2, pair_body, 0)
    wait_out(2 * ((nmine - 1) // 2), 0)

    @pl.when(nmine > 1)
    def _():
        wait_out(nmine - 1 - (nmine % 2), 1)

    if remw:
        @pl.when(wid == 8)
        def _():
            c0 = remc0
            pltpu.sync_copy(t_hbm.at[:, pl.ds(c0, remw)],
                            t0.at[:, pl.ds(0, remw)])
            pltpu.sync_copy(a_hbm.at[:, pl.ds(c0, remw)],
                            a0.at[:, pl.ds(0, remw)])

            @plsc.parallel_loop(0, remw // 16)
            def body(j):
                c = j * 16
                for r in range(16):
                    o0[r, pl.ds(c, 16)] = _tok16(t0[r, pl.ds(c, 16)],
                                                 c0v, c1v)
                for r in range(8):
                    o0[17 + r, pl.ds(c, 16)] = _tok16(a0[r, pl.ds(c, 16)],
                                                      c0v, c1v)
                o0[16, pl.ds(c, 16)] = sep

            pltpu.sync_copy(o0.at[:, pl.ds(0, remw)],
                            o_hbm.at[:, pl.ds(c0, remw)])


def _tok_ref(x):
    mu = jnp.sign(x) * jnp.log(jnp.abs(x) * _MU + 1.0) / np.log(_M * _MU + 1.0)
    v = jnp.floor((jnp.clip(mu, -1.0, 1.0) + 1.0) * (_NB / 2))
    return jnp.clip(v, 0.0, _NB - 1).astype(jnp.int32) + _SHIFT


@jax.jit
def kernel(tensors, actions):
    n = tensors.shape[0]
    nmain = (n // 128) * 128
    nfull = nmain // _CW
    remc0 = nfull * _CW
    remw = nmain - remc0
    mesh = plsc.VectorSubcoreMesh(core_axis_name="c", subcore_axis_name="s")
    run = pl.kernel(
        functools.partial(_sc_body, nfull, remc0, remw),
        out_type=jax.ShapeDtypeStruct((25, n), jnp.int32),
        mesh=mesh,
        compiler_params=pltpu.CompilerParams(needs_layout_passes=False,
                                             use_tc_tiling_on_sc=True),
        scratch_types=[
            pltpu.VMEM((16, _CW), jnp.float32),
            pltpu.VMEM((16, _CW), jnp.float32),
            pltpu.VMEM((8, _CW), jnp.float32),
            pltpu.VMEM((8, _CW), jnp.float32),
            pltpu.VMEM((25, _CW), jnp.int32),
            pltpu.VMEM((25, _CW), jnp.int32),
            pltpu.VMEM((1024,), jnp.float32),
            pltpu.VMEM((1024,), jnp.float32),
            pltpu.SemaphoreType.DMA,
            pltpu.SemaphoreType.DMA,
            pltpu.SemaphoreType.DMA,
            pltpu.SemaphoreType.DMA,
        ],
    )
    out = run(tensors.T, actions.T, jnp.asarray(_T0), jnp.asarray(_T1)).T
    if nmain == n:
        return out
    tt = _tok_ref(tensors[nmain:])
    at = _tok_ref(actions[nmain:])
    sepcol = jnp.full((n - nmain, 1), _SEP, jnp.int32)
    tail = jnp.concatenate([tt, sepcol, at], axis=1)
    return lax.dynamic_update_slice(out, tail, (nmain, 0))
